# baseline (speedup 1.0000x reference)
import functools
import jax
import jax.numpy as jnp
from jax import lax
from jax.experimental import pallas as pl
from jax.experimental.pallas import tpu as pltpu
from jax.experimental.pallas import tpu_sc as plsc

_B = 16384
_C = 26
_D = 64
_NC = 2
_NS = 16
_NW = _NC * _NS
_BPW = _B // _NW
_PASS = 256
_NPASS = _BPW // _PASS
_LANES = 16


def _gather_kernel(x_hbm, label_hbm, out_hbm, lab_v, rows_s, sem):
    cid = lax.axis_index("c")
    sid = lax.axis_index("s")
    wid = sid * _NC + cid
    base = wid * _BPW

    pltpu.sync_copy(label_hbm.at[pl.ds(base, _BPW)], lab_v)

    for p in range(_NPASS):
        pbase = p * _PASS

        def body(i, _):
            r0 = pbase + i * _LANES
            labs = lab_v[pl.ds(r0, _LANES)]
            for u in range(_LANES):
                c = labs[u]
                pltpu.async_copy(
                    x_hbm.at[base + r0 + u, c],
                    rows_s.at[sid, r0 - pbase + u],
                    sem,
                )
            return ()

        lax.fori_loop(0, _PASS // _LANES, body, (), unroll=False)

        pltpu.make_async_copy(
            out_hbm.at[pl.ds(base + pbase, _PASS)], rows_s.at[sid], sem
        ).wait()

        pltpu.sync_copy(rows_s.at[sid], out_hbm.at[pl.ds(base + pbase, _PASS)])


@jax.jit
def _maskout(x, label):
    mesh = plsc.VectorSubcoreMesh(core_axis_name="c", subcore_axis_name="s")
    return pl.kernel(
        _gather_kernel,
        mesh=mesh,
        out_type=jax.ShapeDtypeStruct((_B, _D), jnp.float32),
        scratch_types=[
            pltpu.VMEM((_BPW,), jnp.int32),
            pltpu.VMEM_SHARED((_NS, _PASS, _D), jnp.float32),
            pltpu.SemaphoreType.DMA,
        ],
        compiler_params=pltpu.CompilerParams(use_tc_tiling_on_sc=True),
    )(x, label)


def kernel(x, label):
    return _maskout(x, label.astype(jnp.int32))
